# Initial kernel scaffold; baseline (speedup 1.0000x reference)
import functools

import jax
import jax.numpy as jnp
from jax import lax
from jax.experimental import pallas as pl
from jax.experimental.pallas import tpu as pltpu
from jax.experimental.pallas import tpu_sc as plsc

EMBED = 64
N = 16384 * 50
NC, NS = 2, 16
NW = NC * NS
BPW = N // NW
C = 512
NCHUNK = BPW // C

_mesh = plsc.VectorSubcoreMesh(core_axis_name="c", subcore_axis_name="s")


@functools.partial(
    pl.kernel,
    mesh=_mesh,
    out_type=jax.ShapeDtypeStruct((N, EMBED), jnp.float32),
    scratch_types=[
        pltpu.VMEM((C,), jnp.int32),
        pltpu.VMEM((C, EMBED), jnp.float32),
        pltpu.SemaphoreType.DMA,
    ],
)
def _gather(idx_hbm, table_hbm, out_hbm, idx_v, rows_v, sem):
    wid = lax.axis_index("s") * NC + lax.axis_index("c")
    base = wid * BPW

    def chunk(i, carry):
        off = base + i * C
        pltpu.sync_copy(idx_hbm.at[pl.ds(off, C)], idx_v)
        pltpu.async_copy(table_hbm.at[idx_v], rows_v, sem).wait()
        pltpu.sync_copy(rows_v, out_hbm.at[pl.ds(off, C)])
        return carry

    lax.fori_loop(0, NCHUNK, chunk, 0)


def kernel(indices, table):
    b, l = indices.shape
    out = _gather(indices.reshape(-1), table)
    return out.reshape(b, l, EMBED)


# speedup vs baseline: 4.5066x; 4.5066x over previous
import functools

import jax
import jax.numpy as jnp
from jax import lax
from jax.experimental import pallas as pl
from jax.experimental.pallas import tpu as pltpu
from jax.experimental.pallas import tpu_sc as plsc

EMBED = 64
N = 16384 * 50
NC, NS = 2, 16
NW = NC * NS
BPW = N // NW
G = 4
C = G * 128
NCHUNK = BPW // C

_mesh = plsc.VectorSubcoreMesh(core_axis_name="c", subcore_axis_name="s")


@functools.partial(
    pl.kernel,
    mesh=_mesh,
    out_type=jax.ShapeDtypeStruct((N, 128), jnp.float32),
    scratch_types=[
        pltpu.VMEM((128,), jnp.int32),
        pltpu.VMEM((128, 128), jnp.float32),
        pltpu.VMEM_SHARED((1002, 128), jnp.float32),
        pltpu.SemaphoreType.DMA,
    ],
)
def _gather(idx_hbm, table_hbm, out_hbm, idx_v, rows_v, table_sh, sem):
    sid = lax.axis_index("s")
    wid = sid * NC + lax.axis_index("c")
    base = wid * BPW
    grp_base = wid * (BPW // 128)

    @pl.when(sid == 0)
    def _():
        pltpu.sync_copy(table_hbm, table_sh)

    plsc.subcore_barrier()

    def chunk(i, carry):
        pltpu.sync_copy(idx_hbm.at[grp_base + i], idx_v)
        pltpu.async_copy(table_sh.at[idx_v], rows_v, sem).wait()
        pltpu.sync_copy(rows_v, out_hbm.at[pl.ds(base + i * 128, 128)])
        return carry

    lax.fori_loop(0, BPW // 128, chunk, 0)


def kernel(indices, table):
    b, l = indices.shape
    table_pad = jnp.pad(table, ((0, 0), (0, 128 - EMBED)))
    out = _gather(indices.reshape(N // 128, 128), table_pad)
    return out[:, :EMBED].reshape(b, l, EMBED)


# speedup vs baseline: 6.3102x; 1.4002x over previous
import functools

import jax
import jax.numpy as jnp
from jax import lax
from jax.experimental import pallas as pl
from jax.experimental.pallas import tpu as pltpu
from jax.experimental.pallas import tpu_sc as plsc

B, L, EMBED = 16384, 50, 64
VROWS = 1002
TBL_WORDS = VROWS * EMBED
NC, NS = 2, 16
NW = NC * NS
SPT = B // NW
SPG = 32
NG = SPT // SPG
RING = 4

_mesh = plsc.VectorSubcoreMesh(core_axis_name="c", subcore_axis_name="s")


@functools.partial(
    pl.kernel,
    mesh=_mesh,
    out_type=jax.ShapeDtypeStruct((B, L, EMBED), jnp.float32),
    scratch_types=[
        pltpu.VMEM((TBL_WORDS,), jnp.float32),
        pltpu.VMEM((RING, L, EMBED), jnp.float32),
        pltpu.VMEM((SPG * L + 16,), jnp.int32),
        pltpu.SemaphoreType.DMA,
    ],
)
def _lookup(idx_hbm, tbl_hbm, out_hbm, tbl1, ring, idx_v, sem):
    wid = lax.axis_index("s") * NC + lax.axis_index("c")
    sb = wid * SPT

    pltpu.sync_copy(tbl_hbm, tbl1)

    def group(g, carry):
        pltpu.sync_copy(
            idx_hbm.at[pl.ds((sb + g * SPG) * L, SPG * L)],
            idx_v.at[pl.ds(0, SPG * L)],
        )

        def sample(s, carry2):
            b = sb + g * SPG + s
            slot = lax.rem(s, RING)

            @pl.when(g * SPG + s >= RING)
            def _():
                pltpu.make_async_copy(ring.at[slot], out_hbm.at[b - RING], sem).wait()

            soff = s * L
            ivs = [idx_v[pl.ds(soff + 16 * k, 16)] for k in range(4)]
            for i in range(L):
                a = ivs[i // 16][i % 16]
                for k in range(4):
                    ring[slot, i, pl.ds(k * 16, 16)] = tbl1[pl.ds(a + k * 16, 16)]

            pltpu.async_copy(ring.at[slot], out_hbm.at[b], sem)
            return carry2

        lax.fori_loop(0, SPG, sample, 0)
        return carry

    lax.fori_loop(0, NG, group, 0)

    for k in range(RING):
        pltpu.make_async_copy(ring.at[k], out_hbm.at[sb + SPT - RING + k], sem).wait()


def kernel(indices, table):
    idx64 = (indices.reshape(-1) * EMBED).astype(jnp.int32)
    return _lookup(idx64, table.reshape(-1))
